# baseline (speedup 1.0000x reference)
import functools

import jax
import jax.numpy as jnp
from jax import lax
from jax.experimental import pallas as pl
from jax.experimental.pallas import tpu as pltpu
from jax.experimental.pallas import tpu_sc as plsc

BATCH = 4096
SEQ_LEN = 200
DIM = 64

NUM_CORES = 2
NUM_SUBCORES = 16
NUM_WORKERS = NUM_CORES * NUM_SUBCORES

TOTAL = BATCH * SEQ_LEN
PER_WORKER = TOTAL // NUM_WORKERS
CHUNK = 128
NCHUNK = PER_WORKER // CHUNK
RING = 4
NGROUPS = NCHUNK // RING


@functools.partial(
    pl.kernel,
    mesh=plsc.VectorSubcoreMesh(core_axis_name="c", subcore_axis_name="s"),
    out_type=jax.ShapeDtypeStruct((TOTAL, DIM), jnp.float32),
    scratch_types=[
        pltpu.VMEM((NCHUNK, CHUNK), jnp.int32),
        pltpu.VMEM((RING, CHUNK, DIM), jnp.float32),
        pltpu.SemaphoreType.DMA,
        pltpu.SemaphoreType.DMA,
    ],
    compiler_params=pltpu.CompilerParams(use_tc_tiling_on_sc=False),
)
def _gather_kernel(idx_hbm, table_hbm, out_hbm, idx_v, rows_v, gsem, wsem):
    wid = lax.axis_index("s") * NUM_CORES + lax.axis_index("c")
    pltpu.sync_copy(idx_hbm.at[pl.ds(wid * NCHUNK, NCHUNK)], idx_v)
    base = wid * PER_WORKER

    def group(g, carry):
        j0 = g * RING
        gh = []
        for b in range(RING):
            gh.append(
                pltpu.async_copy(
                    table_hbm.at[idx_v.at[j0 + b]], rows_v.at[b], gsem
                )
            )
        wh = []
        for b in range(RING):
            gh[b].wait()
            wh.append(
                pltpu.async_copy(
                    rows_v.at[b],
                    out_hbm.at[pl.ds(base + (j0 + b) * CHUNK, CHUNK)],
                    wsem,
                )
            )
        for b in range(RING):
            wh[b].wait()
        return carry

    lax.fori_loop(0, NGROUPS, group, 0)


def kernel(token_ids, weight):
    flat_idx = token_ids.reshape(NUM_WORKERS * NCHUNK, CHUNK)
    out = _gather_kernel(flat_idx, weight)
    return out.reshape(BATCH, SEQ_LEN, DIM)


# speedup vs baseline: 1.2262x; 1.2262x over previous
import functools

import jax
import jax.numpy as jnp
from jax import lax
from jax.experimental import pallas as pl
from jax.experimental.pallas import tpu as pltpu
from jax.experimental.pallas import tpu_sc as plsc

BATCH = 4096
SEQ_LEN = 200
DIM = 64
PDIM = 128

NUM_CORES = 2
NUM_SUBCORES = 16
NUM_WORKERS = NUM_CORES * NUM_SUBCORES

TOTAL = BATCH * SEQ_LEN
PER_WORKER = TOTAL // NUM_WORKERS
CHUNK = 128
NCHUNK = PER_WORKER // CHUNK
RING = 4
NGROUPS = NCHUNK // RING

NUM_EMB = 1000000


@functools.partial(
    pl.kernel,
    mesh=plsc.VectorSubcoreMesh(core_axis_name="c", subcore_axis_name="s"),
    out_type=jax.ShapeDtypeStruct((TOTAL, PDIM), jnp.float32),
    scratch_types=[
        pltpu.VMEM((NCHUNK, CHUNK), jnp.int32),
        pltpu.VMEM((RING, CHUNK, PDIM), jnp.float32),
        pltpu.SemaphoreType.DMA,
        pltpu.SemaphoreType.DMA,
    ],
)
def _gather_kernel(idx_hbm, table_hbm, out_hbm, idx_v, rows_v, gsem, wsem):
    wid = lax.axis_index("s") * NUM_CORES + lax.axis_index("c")
    pltpu.sync_copy(idx_hbm.at[pl.ds(wid * NCHUNK, NCHUNK)], idx_v)
    base = wid * PER_WORKER

    table_rows = table_hbm
    out_rows = out_hbm

    def group(g, carry):
        j0 = g * RING
        gh = []
        for b in range(RING):
            gh.append(
                pltpu.async_copy(
                    table_rows.at[idx_v.at[j0 + b]], rows_v.at[b], gsem
                )
            )
        wh = []
        for b in range(RING):
            gh[b].wait()
            wh.append(
                pltpu.async_copy(
                    rows_v.at[b],
                    out_rows.at[pl.ds(base + (j0 + b) * CHUNK, CHUNK)],
                    wsem,
                )
            )
        for b in range(RING):
            wh[b].wait()
        return carry

    lax.fori_loop(0, NGROUPS, group, 0)


def kernel(token_ids, weight):
    wpad = jnp.pad(weight, ((0, 0), (0, PDIM - DIM)))
    flat_idx = token_ids.reshape(NUM_WORKERS * NCHUNK, CHUNK)
    out = _gather_kernel(flat_idx, wpad)
    return out[:, :DIM].reshape(BATCH, SEQ_LEN, DIM)
